# baseline (speedup 1.0000x reference)
import functools

import jax
import jax.numpy as jnp
from jax import lax
from jax.experimental import pallas as pl
from jax.experimental.pallas import tpu as pltpu
from jax.experimental.pallas import tpu_sc as plsc

BATCH = 16384
VOCAB = 100000
EMB = 128
CTX = 6
OUT = 91

NUM_WORKERS = 32
NBUF = 3
CHUNK_SIZES = (8192, 6144, 2048)
CHUNK_OFFS = (0, 8192, 14336)
STREAM_ROWS = (256, 128, 128)


def _gather_body(off, bch, upw, CHUNK, table_hbm, idxT_hbm, out_hbm, idx_v, *rest):
    spans = bch // CHUNK
    bufs = rest[:NBUF]
    gsems = rest[NBUF:2 * NBUF]
    wsems = rest[2 * NBUF:3 * NBUF]
    isem = rest[3 * NBUF]
    wid = lax.axis_index("s") * 2 + lax.axis_index("c")
    u0 = wid * upw
    js = []
    ics = []
    for t in range(upw):
        u = u0 + t
        j = u // spans
        s = u % spans
        js.append((j, s))
        ics.append(pltpu.async_copy(
            idxT_hbm.at[j, pl.ds(off + s * CHUNK, CHUNK)],
            idx_v.at[pl.ds(t * CHUNK, CHUNK)], isem))
    for ic in ics:
        ic.wait()

    def src(t):
        return table_hbm.at[idx_v.at[pl.ds(t * CHUNK, CHUNK)]]

    def dst(t):
        j, s = js[t]
        return out_hbm.at[pl.ds(j * bch + s * CHUNK, CHUNK)]

    gcs = [None] * NBUF
    wcs = [None] * NBUF
    for t in range(min(NBUF, upw)):
        gcs[t] = pltpu.async_copy(src(t), bufs[t], gsems[t])
    for c in range(upw):
        k = c % NBUF
        gcs[k].wait()
        wcs[k] = pltpu.async_copy(bufs[k], dst(c), wsems[k])
        if c + NBUF < upw:
            wcs[k].wait()
            gcs[k] = pltpu.async_copy(src(c + NBUF), bufs[k], gsems[k])
    for c in range(max(0, upw - NBUF), upw):
        wcs[c % NBUF].wait()


def _make_gather(off, bch, chunk):
    upw = CTX * (bch // chunk) // NUM_WORKERS
    return pl.kernel(
        functools.partial(_gather_body, off, bch, upw, chunk),
        out_type=jax.ShapeDtypeStruct((CTX * bch, EMB), jnp.float32),
        mesh=plsc.VectorSubcoreMesh(core_axis_name="c", subcore_axis_name="s"),
        scratch_types=(
            [pltpu.VMEM((upw * chunk,), jnp.int32)]
            + [pltpu.VMEM((chunk, EMB), jnp.float32) for _ in range(NBUF)]
            + [pltpu.SemaphoreType.DMA for _ in range(2 * NBUF + 1)]
        ),
    )


_gathers = [_make_gather(o, b, s)
            for o, b, s in zip(CHUNK_OFFS, CHUNK_SIZES, STREAM_ROWS)]


def _mlp_body(x_ref, w1_ref, b1_ref, w2t_ref, b2_ref, *refs):
    out_ref = refs[-1]
    h = jnp.broadcast_to(b1_ref[...].astype(jnp.float32), (x_ref.shape[1], EMB))
    for j in range(CTX):
        h = h + jax.lax.dot_general(
            x_ref[j], w1_ref[j], (((1,), (0,)), ((), ())),
            preferred_element_type=jnp.float32)
    h = jnp.maximum(h, 0.0)
    logits_t = jax.lax.dot_general(w2t_ref[...], h, (((1,), (1,)), ((), ())),
                                   preferred_element_type=jnp.float32) + b2_ref[...]
    m = jnp.max(logits_t, axis=0, keepdims=True)
    s = logits_t - m
    lse = jnp.log(jnp.sum(jnp.exp(s), axis=0, keepdims=True))
    out_ref[...] = s - lse


BLOCK_B = 1024


def _mlp_chunk(off, bch, x3, W1r, b1, W2t, b2, prev=None):
    grid = (bch // BLOCK_B,)
    blk0 = off // BLOCK_B
    in_specs = [
        pl.BlockSpec((CTX, BLOCK_B, EMB), lambda i: (0, i, 0)),
        pl.BlockSpec((CTX, EMB, EMB), lambda i: (0, 0, 0)),
        pl.BlockSpec((1, EMB), lambda i: (0, 0)),
        pl.BlockSpec((OUT, EMB), lambda i: (0, 0)),
        pl.BlockSpec((OUT, 1), lambda i: (0, 0)),
    ]
    args = [x3, W1r, b1, W2t, b2]
    aliases = {}
    if prev is not None:
        in_specs.append(pl.BlockSpec((OUT, BLOCK_B), lambda i: (0, i + blk0)))
        args.append(prev)
        aliases = {5: 0}
    return pl.pallas_call(
        _mlp_body,
        grid=grid,
        in_specs=in_specs,
        out_specs=pl.BlockSpec((OUT, BLOCK_B), lambda i: (0, i + blk0)),
        out_shape=jax.ShapeDtypeStruct((OUT, BATCH), jnp.float32),
        input_output_aliases=aliases,
    )(*args)


@jax.jit
def kernel(inputs, emb, W1, b1, W2, b2):
    idxT = jnp.transpose(inputs)
    W1r = W1.reshape(CTX, EMB, EMB)
    b1r = b1.reshape(1, EMB)
    W2t = jnp.transpose(W2)
    b2r = b2.reshape(OUT, 1)
    out = None
    for c, (off, bch) in enumerate(zip(CHUNK_OFFS, CHUNK_SIZES)):
        g = _gathers[c](emb, idxT)
        x3 = g.reshape(CTX, bch, EMB)
        out = _mlp_chunk(off, bch, x3, W1r, b1r, W2t, b2r, out)
    return jnp.transpose(out)


# speedup vs baseline: 1.0272x; 1.0272x over previous
import functools

import jax
import jax.numpy as jnp
from jax import lax
from jax.experimental import pallas as pl
from jax.experimental.pallas import tpu as pltpu
from jax.experimental.pallas import tpu_sc as plsc

BATCH = 16384
VOCAB = 100000
EMB = 128
CTX = 6
OUT = 91

NUM_WORKERS = 32
NBUF = 3
CHUNK_SIZES = (8192, 4096, 4096)
CHUNK_OFFS = (0, 8192, 12288)
STREAM_ROWS = (256, 256, 256)


def _gather_body(off, bch, upw, CHUNK, table_hbm, idxT_hbm, out_hbm, idx_v, *rest):
    spans = bch // CHUNK
    bufs = rest[:NBUF]
    gsems = rest[NBUF:2 * NBUF]
    wsems = rest[2 * NBUF:3 * NBUF]
    isem = rest[3 * NBUF]
    wid = lax.axis_index("s") * 2 + lax.axis_index("c")
    u0 = wid * upw
    js = []
    ics = []
    for t in range(upw):
        u = u0 + t
        j = u // spans
        s = u % spans
        js.append((j, s))
        ics.append(pltpu.async_copy(
            idxT_hbm.at[j, pl.ds(off + s * CHUNK, CHUNK)],
            idx_v.at[pl.ds(t * CHUNK, CHUNK)], isem))
    for ic in ics:
        ic.wait()

    def src(t):
        return table_hbm.at[idx_v.at[pl.ds(t * CHUNK, CHUNK)]]

    def dst(t):
        j, s = js[t]
        return out_hbm.at[pl.ds(j * bch + s * CHUNK, CHUNK)]

    gcs = [None] * NBUF
    wcs = [None] * NBUF
    for t in range(min(NBUF, upw)):
        gcs[t] = pltpu.async_copy(src(t), bufs[t], gsems[t])
    for c in range(upw):
        k = c % NBUF
        gcs[k].wait()
        wcs[k] = pltpu.async_copy(bufs[k], dst(c), wsems[k])
        if c + NBUF < upw:
            wcs[k].wait()
            gcs[k] = pltpu.async_copy(src(c + NBUF), bufs[k], gsems[k])
    for c in range(max(0, upw - NBUF), upw):
        wcs[c % NBUF].wait()


def _make_gather(off, bch, chunk):
    upw = CTX * (bch // chunk) // NUM_WORKERS
    return pl.kernel(
        functools.partial(_gather_body, off, bch, upw, chunk),
        out_type=jax.ShapeDtypeStruct((CTX * bch, EMB), jnp.float32),
        mesh=plsc.VectorSubcoreMesh(core_axis_name="c", subcore_axis_name="s"),
        scratch_types=(
            [pltpu.VMEM((upw * chunk,), jnp.int32)]
            + [pltpu.VMEM((chunk, EMB), jnp.float32) for _ in range(NBUF)]
            + [pltpu.SemaphoreType.DMA for _ in range(2 * NBUF + 1)]
        ),
    )


_gathers = [_make_gather(o, b, s)
            for o, b, s in zip(CHUNK_OFFS, CHUNK_SIZES, STREAM_ROWS)]


def _mlp_body(x_ref, w1_ref, b1_ref, w2t_ref, b2_ref, *refs):
    out_ref = refs[-1]
    h = jnp.broadcast_to(b1_ref[...].astype(jnp.float32), (x_ref.shape[1], EMB))
    for j in range(CTX):
        h = h + jax.lax.dot_general(
            x_ref[j], w1_ref[j], (((1,), (0,)), ((), ())),
            preferred_element_type=jnp.float32)
    h = jnp.maximum(h, 0.0)
    logits_t = jax.lax.dot_general(w2t_ref[...], h, (((1,), (1,)), ((), ())),
                                   preferred_element_type=jnp.float32) + b2_ref[...]
    m = jnp.max(logits_t, axis=0, keepdims=True)
    s = logits_t - m
    lse = jnp.log(jnp.sum(jnp.exp(s), axis=0, keepdims=True))
    out_ref[...] = s - lse


BLOCK_B = 1024


def _mlp_chunk(off, bch, x3, W1r, b1, W2t, b2, prev=None):
    grid = (bch // BLOCK_B,)
    blk0 = off // BLOCK_B
    in_specs = [
        pl.BlockSpec((CTX, BLOCK_B, EMB), lambda i: (0, i, 0)),
        pl.BlockSpec((CTX, EMB, EMB), lambda i: (0, 0, 0)),
        pl.BlockSpec((1, EMB), lambda i: (0, 0)),
        pl.BlockSpec((OUT, EMB), lambda i: (0, 0)),
        pl.BlockSpec((OUT, 1), lambda i: (0, 0)),
    ]
    args = [x3, W1r, b1, W2t, b2]
    aliases = {}
    if prev is not None:
        in_specs.append(pl.BlockSpec((OUT, BLOCK_B), lambda i: (0, i + blk0)))
        args.append(prev)
        aliases = {5: 0}
    return pl.pallas_call(
        _mlp_body,
        grid=grid,
        in_specs=in_specs,
        out_specs=pl.BlockSpec((OUT, BLOCK_B), lambda i: (0, i + blk0)),
        out_shape=jax.ShapeDtypeStruct((OUT, BATCH), jnp.float32),
        input_output_aliases=aliases,
    )(*args)


@jax.jit
def kernel(inputs, emb, W1, b1, W2, b2):
    idxT = jnp.transpose(inputs)
    W1r = W1.reshape(CTX, EMB, EMB)
    b1r = b1.reshape(1, EMB)
    W2t = jnp.transpose(W2)
    b2r = b2.reshape(OUT, 1)
    out = None
    for c, (off, bch) in enumerate(zip(CHUNK_OFFS, CHUNK_SIZES)):
        g = _gathers[c](emb, idxT)
        x3 = g.reshape(CTX, bch, EMB)
        out = _mlp_chunk(off, bch, x3, W1r, b1r, W2t, b2r, out)
    return jnp.transpose(out)


# speedup vs baseline: 1.0324x; 1.0050x over previous
import functools

import jax
import jax.numpy as jnp
from jax import lax
from jax.experimental import pallas as pl
from jax.experimental.pallas import tpu as pltpu
from jax.experimental.pallas import tpu_sc as plsc

BATCH = 16384
VOCAB = 100000
EMB = 128
CTX = 6
OUT = 91

NUM_WORKERS = 32
NBUF = 3
CHUNK_SIZES = (8192, 4096, 4096)
CHUNK_OFFS = (0, 8192, 12288)
STREAM_ROWS = (256, 256, 256)


def _gather_body(off, bch, upw, CHUNK, table_hbm, idxT_hbm, out_hbm, idx_v, *rest):
    spans = bch // CHUNK
    bufs = rest[:NBUF]
    gsems = rest[NBUF:2 * NBUF]
    wsems = rest[2 * NBUF:3 * NBUF]
    isem = rest[3 * NBUF]
    wid = lax.axis_index("s") * 2 + lax.axis_index("c")
    u0 = wid * upw
    js = []
    ics = []
    for t in range(upw):
        u = u0 + t
        j = u // spans
        s = u % spans
        js.append((j, s))
        ics.append(pltpu.async_copy(
            idxT_hbm.at[j, pl.ds(off + s * CHUNK, CHUNK)],
            idx_v.at[pl.ds(t * CHUNK, CHUNK)], isem))
    for ic in ics:
        ic.wait()

    def src(t):
        return table_hbm.at[idx_v.at[pl.ds(t * CHUNK, CHUNK)]]

    def dst(t):
        j, s = js[t]
        return out_hbm.at[pl.ds(j * bch + s * CHUNK, CHUNK)]

    del wsems
    gcs = [None] * NBUF
    for t in range(min(NBUF, upw)):
        gcs[t] = pltpu.async_copy(src(t), bufs[t], gsems[t])
    for c in range(upw):
        k = c % NBUF
        gcs[k].wait()
        pltpu.sync_copy(bufs[k], dst(c))
        if c + NBUF < upw:
            gcs[k] = pltpu.async_copy(src(c + NBUF), bufs[k], gsems[k])


def _make_gather(off, bch, chunk):
    upw = CTX * (bch // chunk) // NUM_WORKERS
    return pl.kernel(
        functools.partial(_gather_body, off, bch, upw, chunk),
        out_type=jax.ShapeDtypeStruct((CTX * bch, EMB), jnp.float32),
        mesh=plsc.VectorSubcoreMesh(core_axis_name="c", subcore_axis_name="s"),
        scratch_types=(
            [pltpu.VMEM((upw * chunk,), jnp.int32)]
            + [pltpu.VMEM((chunk, EMB), jnp.float32) for _ in range(NBUF)]
            + [pltpu.SemaphoreType.DMA for _ in range(2 * NBUF + 1)]
        ),
    )


_gathers = [_make_gather(o, b, s)
            for o, b, s in zip(CHUNK_OFFS, CHUNK_SIZES, STREAM_ROWS)]


def _mlp_body(x_ref, w1_ref, b1_ref, w2t_ref, b2_ref, *refs):
    out_ref = refs[-1]
    h = jnp.broadcast_to(b1_ref[...].astype(jnp.float32), (x_ref.shape[1], EMB))
    for j in range(CTX):
        h = h + jax.lax.dot_general(
            x_ref[j], w1_ref[j], (((1,), (0,)), ((), ())),
            preferred_element_type=jnp.float32)
    h = jnp.maximum(h, 0.0)
    logits_t = jax.lax.dot_general(w2t_ref[...], h, (((1,), (1,)), ((), ())),
                                   preferred_element_type=jnp.float32) + b2_ref[...]
    m = jnp.max(logits_t, axis=0, keepdims=True)
    s = logits_t - m
    lse = jnp.log(jnp.sum(jnp.exp(s), axis=0, keepdims=True))
    out_ref[...] = s - lse


BLOCK_B = 1024


def _mlp_chunk(off, bch, x3, W1r, b1, W2t, b2, prev=None):
    grid = (bch // BLOCK_B,)
    blk0 = off // BLOCK_B
    in_specs = [
        pl.BlockSpec((CTX, BLOCK_B, EMB), lambda i: (0, i, 0)),
        pl.BlockSpec((CTX, EMB, EMB), lambda i: (0, 0, 0)),
        pl.BlockSpec((1, EMB), lambda i: (0, 0)),
        pl.BlockSpec((OUT, EMB), lambda i: (0, 0)),
        pl.BlockSpec((OUT, 1), lambda i: (0, 0)),
    ]
    args = [x3, W1r, b1, W2t, b2]
    aliases = {}
    if prev is not None:
        in_specs.append(pl.BlockSpec((OUT, BLOCK_B), lambda i: (0, i + blk0)))
        args.append(prev)
        aliases = {5: 0}
    return pl.pallas_call(
        _mlp_body,
        grid=grid,
        in_specs=in_specs,
        out_specs=pl.BlockSpec((OUT, BLOCK_B), lambda i: (0, i + blk0)),
        out_shape=jax.ShapeDtypeStruct((OUT, BATCH), jnp.float32),
        input_output_aliases=aliases,
    )(*args)


@jax.jit
def kernel(inputs, emb, W1, b1, W2, b2):
    idxT = jnp.transpose(inputs)
    W1r = W1.reshape(CTX, EMB, EMB)
    b1r = b1.reshape(1, EMB)
    W2t = jnp.transpose(W2)
    b2r = b2.reshape(OUT, 1)
    out = None
    for c, (off, bch) in enumerate(zip(CHUNK_OFFS, CHUNK_SIZES)):
        g = _gathers[c](emb, idxT)
        x3 = g.reshape(CTX, bch, EMB)
        out = _mlp_chunk(off, bch, x3, W1r, b1r, W2t, b2r, out)
    return jnp.transpose(out)
